# baseline (speedup 1.0000x reference)
import functools

import jax
import jax.numpy as jnp
from jax import lax
from jax.experimental import pallas as pl
from jax.experimental.pallas import tpu as pltpu
from jax.experimental.pallas import tpu_sc as plsc

_SEED_ROWS = 64


def _sc_seed(logs, V):
    info = plsc.get_sparse_core_info()
    NC, NS = info.num_cores, info.num_subcores
    NW = NC * NS
    rows_per_w = _SEED_ROWS // NW

    mesh = plsc.VectorSubcoreMesh(core_axis_name="c", subcore_axis_name="s")

    @functools.partial(
        pl.kernel,
        mesh=mesh,
        out_type=jax.ShapeDtypeStruct((_SEED_ROWS, V), jnp.float32),
        scratch_types=[
            pltpu.VMEM((V,), jnp.float32),
            pltpu.SemaphoreType.DMA,
        ],
    )
    def seed_kernel(logs_hbm, seed_hbm, row_v, sem):
        wid = lax.axis_index("s") * NC + lax.axis_index("c")
        pltpu.sync_copy(logs_hbm.at[pl.ds(0, V)], row_v)
        base = wid * rows_per_w
        copies = [
            pltpu.make_async_copy(row_v, seed_hbm.at[base + i], sem)
            for i in range(rows_per_w)
        ]
        for c in copies:
            c.start()
        for c in copies:
            c.wait()

    return seed_kernel(logs)


def _tc_expand(seed, B, V):
    nblk = B // _SEED_ROWS

    def body(seed_vmem, out_hbm, sem):
        copies = [
            pltpu.make_async_copy(
                seed_vmem, out_hbm.at[pl.ds(k * _SEED_ROWS, _SEED_ROWS)], sem
            )
            for k in range(nblk)
        ]
        for c in copies:
            c.start()
        for c in copies:
            c.wait()

    return pl.pallas_call(
        body,
        in_specs=[pl.BlockSpec(memory_space=pltpu.VMEM)],
        out_specs=pl.BlockSpec(memory_space=pl.ANY),
        out_shape=jax.ShapeDtypeStruct((B, V), jnp.float32),
        scratch_shapes=[pltpu.SemaphoreType.DMA],
    )(seed)


def kernel(hist, idx, logs):
    B = hist.shape[1]
    V = logs.shape[0] - 1
    seed = _sc_seed(logs, V)
    return _tc_expand(seed, B, V)
